# baseline (speedup 1.0000x reference)
import functools

import jax
import jax.numpy as jnp
from jax import lax
from jax.experimental import pallas as pl
from jax.experimental.pallas import tpu as pltpu
from jax.experimental.pallas import tpu_sc as plsc

_NC = 2
_NS = 16
_CHUNK = 128


def _repack_body(lo, hi, out):
    out[...] = jnp.concatenate([lo[...].T, hi[...].T], axis=1)


def _repack(tabT, bm=8192):
    E, V = tabT.shape
    nlo = pl.cdiv((V + 1) // 2, bm)
    Vh = nlo * bm
    last = pl.cdiv(V, bm) - 1
    return pl.pallas_call(
        _repack_body,
        grid=(nlo,),
        in_specs=[
            pl.BlockSpec((E, bm), lambda i: (0, i)),
            pl.BlockSpec((E, bm), lambda i: (0, jnp.minimum(i + nlo, last))),
        ],
        out_specs=pl.BlockSpec((bm, 2 * E), lambda i: (i, 0)),
        out_shape=jax.ShapeDtypeStruct((Vh, 2 * E), jnp.float32),
        compiler_params=pltpu.CompilerParams(
            dimension_semantics=("arbitrary",)),
    )(tabT, tabT)


def _sc_gather(idx2, tutor_pair, time_pad, B, E):
    nw = _NC * _NS
    bpw = B // nw
    nch = bpw // _CHUNK
    nrows = B // _CHUNK

    mesh = plsc.VectorSubcoreMesh(
        core_axis_name="c", subcore_axis_name="s",
        num_cores=_NC, num_subcores=_NS)

    @functools.partial(
        pl.kernel,
        mesh=mesh,
        compiler_params=pltpu.CompilerParams(use_tc_tiling_on_sc=True),
        out_type=jax.ShapeDtypeStruct((B, 4 * E), jnp.float32),
        scratch_types=[
            pltpu.VMEM((nch, _CHUNK), jnp.int32),
            pltpu.VMEM((nch, _CHUNK), jnp.int32),
            pltpu.VMEM((bpw, 2 * E), jnp.float32),
            pltpu.VMEM((bpw // 2, 2 * E), jnp.float32),
            pltpu.SemaphoreType.DMA,
            pltpu.SemaphoreType.DMA,
        ],
    )
    def gather_kernel(idx_hbm, ttab_hbm, mtab_hbm, out_hbm,
                      tidx_v, midx_v, trows_v, mrows_v, tsem, msem):
        wid = lax.axis_index("s") * _NC + lax.axis_index("c")
        base = wid * bpw
        half = bpw // 2
        lanes_t = pl.ds(0, 2 * E)
        lanes_m = pl.ds(2 * E, 2 * E)
        pltpu.sync_copy(idx_hbm.at[pl.ds(wid * nch, nch)], tidx_v)
        pltpu.sync_copy(idx_hbm.at[pl.ds(nrows + wid * nch, nch)], midx_v)
        tcopies = []
        for j in range(nch):
            tcopies.append(pltpu.async_copy(
                ttab_hbm.at[tidx_v.at[j]], trows_v.at[pl.ds(j * _CHUNK, _CHUNK)],
                tsem))
        for p in range(2):
            mcopies = []
            for j in range(nch // 2):
                mcopies.append(pltpu.async_copy(
                    mtab_hbm.at[midx_v.at[p * (nch // 2) + j]],
                    mrows_v.at[pl.ds(j * _CHUNK, _CHUNK)], msem))
            for c in mcopies:
                c.wait()
            pltpu.sync_copy(mrows_v, out_hbm.at[pl.ds(base + p * half, half), lanes_m])
        for c in tcopies:
            c.wait()
        pltpu.sync_copy(trows_v, out_hbm.at[pl.ds(base, bpw), lanes_t])

    return gather_kernel(idx2, tutor_pair, time_pad)


def _mlp_body(emb, parity, featT, wlo, whi, wm, wblk, bsml,
              w1, b1, w2, b2, w3, b3, outT):
    f32 = jnp.float32
    bf16 = jnp.bfloat16
    small = lax.dot_general(
        featT[...], wblk[...], (((0,), (0,)), ((), ())),
        preferred_element_type=f32) + bsml[...]
    e = emb[...]
    et = e[:, 0:128].astype(bf16)
    em = e[:, 128:256].astype(bf16)
    h_lo = jnp.dot(et, wlo[...].astype(bf16), preferred_element_type=f32)
    h_hi = jnp.dot(et, whi[...].astype(bf16), preferred_element_type=f32)
    h_t = jnp.where(parity[...] > 0.5, h_hi, h_lo)
    h = (h_t
         + jnp.dot(em, wm[...].astype(bf16), preferred_element_type=f32)
         + jnp.dot(small.astype(bf16), w1[128:224, :].astype(bf16),
                   preferred_element_type=f32)
         + b1[...])
    h = jnp.maximum(h, 0.0)
    h = jnp.maximum(
        jnp.dot(h.astype(bf16), w2[...].astype(bf16),
                preferred_element_type=f32) + b2[...], 0.0)
    out = jnp.dot(h.astype(bf16), w3[...].astype(bf16),
                  preferred_element_type=f32) + b3[...]
    outT[...] = out.T


def _mlp(emb, parity, featT, wlo, whi, wm, wblk, bsml,
         W1, b1, W2, b2, W3, b3, bm=2048):
    B = emb.shape[0]
    grid = (B // bm,)
    no = W3.shape[1]

    in_specs = [
        pl.BlockSpec((bm, emb.shape[1]), lambda i: (i, 0)),
        pl.BlockSpec((bm, 1), lambda i: (i, 0)),
        pl.BlockSpec((featT.shape[0], bm), lambda i: (0, i)),
        pl.BlockSpec(wlo.shape, lambda i: (0, 0)),
        pl.BlockSpec(whi.shape, lambda i: (0, 0)),
        pl.BlockSpec(wm.shape, lambda i: (0, 0)),
        pl.BlockSpec(wblk.shape, lambda i: (0, 0)),
        pl.BlockSpec(bsml.shape, lambda i: (0, 0)),
        pl.BlockSpec(W1.shape, lambda i: (0, 0)),
        pl.BlockSpec(b1.shape, lambda i: (0, 0)),
        pl.BlockSpec(W2.shape, lambda i: (0, 0)),
        pl.BlockSpec(b2.shape, lambda i: (0, 0)),
        pl.BlockSpec(W3.shape, lambda i: (0, 0)),
        pl.BlockSpec(b3.shape, lambda i: (0, 0)),
    ]
    return pl.pallas_call(
        _mlp_body,
        grid=grid,
        in_specs=in_specs,
        out_specs=pl.BlockSpec((no, bm), lambda i: (0, i)),
        out_shape=jax.ShapeDtypeStruct((no, B), jnp.float32),
        compiler_params=pltpu.CompilerParams(
            dimension_semantics=("arbitrary",)),
    )(emb, parity, featT, wlo, whi, wm, wblk, bsml,
      W1, b1, W2, b2, W3, b3)


def kernel(tutor_idx, time_idx, experience, subject_pca, grade_pca,
           tutor_table, time_table, Ws, bs, Wg, bg, We, be,
           W1, b1, W2, b2, W3, b3):
    B = tutor_idx.shape[0]
    E = tutor_table.shape[1]

    V = tutor_table.shape[0]
    Vh = pl.cdiv((V + 1) // 2, 8192) * 8192
    trow = jnp.where(tutor_idx < Vh, tutor_idx, tutor_idx - Vh)
    idx2 = jnp.concatenate([trow, time_idx]).reshape(2 * B // _CHUNK, _CHUNK)
    parity = (tutor_idx >= Vh).astype(jnp.float32)[:, None]

    tutor_pair = _repack(tutor_table.T)
    time_pad = jnp.pad(time_table, ((0, 0), (0, E)))
    emb = _sc_gather(idx2, tutor_pair, time_pad, B, E)

    featT = jnp.concatenate(
        [subject_pca.T, grade_pca.T, experience[None, :]], axis=0)
    z = jnp.zeros
    f32 = jnp.float32
    wblk = jnp.concatenate([
        jnp.concatenate([Ws, z((10, 64), f32)], axis=1),
        jnp.concatenate([z((5, 32), f32), Wg, z((5, 32), f32)], axis=1),
        jnp.concatenate([z((1, 64), f32), We], axis=1),
    ], axis=0)
    bsml = jnp.concatenate([bs, bg, be])[None, :]
    n1 = W1.shape[1]
    wlo = jnp.concatenate([W1[0:E, :], z((E, n1), f32)], axis=0)
    whi = jnp.concatenate([z((E, n1), f32), W1[0:E, :]], axis=0)
    wm = jnp.concatenate([W1[E:2 * E, :], z((E, n1), f32)], axis=0)

    outT = _mlp(emb, parity, featT, wlo, whi, wm, wblk, bsml,
                W1, b1[None, :], W2, b2[None, :], W3, b3[None, :])
    return outT.T
